# Initial kernel scaffold; baseline (speedup 1.0000x reference)
import jax
import jax.numpy as jnp
from jax.experimental import pallas as pl


def kernel(ecg_tokens, emb_table):
    raise NotImplementedError("write your pallas kernel here")


# speedup vs baseline: 1.8152x; 1.8152x over previous
import functools

import jax
import jax.numpy as jnp
from jax import lax
from jax.experimental import pallas as pl
from jax.experimental.pallas import tpu as pltpu
from jax.experimental.pallas import tpu_sc as plsc

PAD = -2


@functools.lru_cache(maxsize=None)
def _build(N, V, D):
    info = plsc.get_sparse_core_info()
    NC, NS, LANES = info.num_cores, info.num_subcores, info.num_lanes
    NW = NC * NS
    assert N % NW == 0
    per_w = N // NW
    K = 640
    assert per_w % K == 0
    nblk = per_w // K
    GCH = 128
    ng = K // GCH

    mesh = plsc.VectorSubcoreMesh(core_axis_name="c", subcore_axis_name="s")

    @functools.partial(
        pl.kernel,
        out_type=jax.ShapeDtypeStruct((N, D), jnp.float32),
        mesh=mesh,
        compiler_params=pltpu.CompilerParams(
            needs_layout_passes=False, use_tc_tiling_on_sc=False
        ),
        scratch_types=[
            pltpu.VMEM((K,), jnp.int32),
            pltpu.VMEM((ng, GCH), jnp.int32),
            pltpu.VMEM((K, D), jnp.float32),
            pltpu.SemaphoreType.DMA,
        ],
    )
    def k(tok_hbm, table_hbm, out_hbm, tok_v, idx_v, rows_v, sem):
        wid = lax.axis_index("s") * NC + lax.axis_index("c")
        lane = lax.iota(jnp.int32, LANES)
        zeros = jnp.zeros((LANES,), jnp.float32)

        def block(g, _):
            base = wid * per_w + g * K
            pltpu.sync_copy(tok_hbm.at[pl.ds(base, K)], tok_v)

            for i in range(K // LANES):
                v = tok_v[pl.ds(i * LANES, LANES)]
                r, c = (i * LANES) // GCH, (i * LANES) % GCH
                idx_v[r, pl.ds(c, LANES)] = jnp.maximum(v, 0)

            copies = [
                pltpu.async_copy(
                    table_hbm.at[idx_v.at[j]],
                    rows_v.at[pl.ds(j * GCH, GCH)],
                    sem,
                )
                for j in range(ng)
            ]
            for cp in copies:
                cp.wait()

            def zero_group(j, _):
                v = tok_v[pl.ds(j * LANES, LANES)]
                m = v < 0
                rowids = j * LANES + lane
                for col in range(D):
                    cvec = jnp.full((LANES,), col, jnp.int32)
                    plsc.store_scatter(rows_v, [rowids, cvec], zeros, mask=m)
                return 0

            lax.fori_loop(0, K // LANES, zero_group, 0)

            pltpu.sync_copy(rows_v, out_hbm.at[pl.ds(base, K)])
            return 0

        lax.fori_loop(0, nblk, block, 0)

    return k


def kernel(ecg_tokens, emb_table):
    B, L = ecg_tokens.shape
    V, D = emb_table.shape
    N = B * L
    k = _build(N, V, D)
    out = k(ecg_tokens.reshape(N), emb_table)
    return out.reshape(B, L, D)


# speedup vs baseline: 1.8171x; 1.0010x over previous
import functools

import jax
import jax.numpy as jnp
from jax import lax
from jax.experimental import pallas as pl
from jax.experimental.pallas import tpu as pltpu
from jax.experimental.pallas import tpu_sc as plsc

PAD = -2


@functools.lru_cache(maxsize=None)
def _build(N, V, D):
    info = plsc.get_sparse_core_info()
    NC, NS, LANES = info.num_cores, info.num_subcores, info.num_lanes
    NW = NC * NS
    assert N % NW == 0
    per_w = N // NW
    GCH = 128
    assert per_w % GCH == 0
    nb = per_w // GCH
    R = 8
    Q = 6

    mesh = plsc.VectorSubcoreMesh(core_axis_name="c", subcore_axis_name="s")

    @functools.partial(
        pl.kernel,
        out_type=jax.ShapeDtypeStruct((N, D), jnp.float32),
        mesh=mesh,
        compiler_params=pltpu.CompilerParams(
            needs_layout_passes=False, use_tc_tiling_on_sc=False
        ),
        scratch_types=[
            pltpu.VMEM((per_w,), jnp.int32),
            pltpu.VMEM((nb, GCH), jnp.int32),
            pltpu.VMEM((R * GCH, D), jnp.float32),
            pltpu.SemaphoreType.DMA,
            pltpu.SemaphoreType.DMA,
        ],
    )
    def k(tok_hbm, table_hbm, out_hbm, tok_v, idx_v, rows_v, sem_g, sem_o):
        wid = lax.axis_index("s") * NC + lax.axis_index("c")
        base = wid * per_w
        lane = lax.iota(jnp.int32, LANES)
        zeros = jnp.zeros((LANES,), jnp.float32)
        cols = [jnp.full((LANES,), c, jnp.int32) for c in range(D)]

        pltpu.sync_copy(tok_hbm.at[pl.ds(base, per_w)], tok_v)

        def prep(i2, _):
            for u in range(GCH // LANES):
                v = tok_v[pl.ds(i2 * GCH + u * LANES, LANES)]
                idx_v[i2, pl.ds(u * LANES, LANES)] = jnp.maximum(v, 0)
            return 0

        lax.fori_loop(0, nb, prep, 0)

        def g_desc(j):
            s = lax.rem(j, R)
            return pltpu.make_async_copy(
                table_hbm.at[idx_v.at[j]],
                rows_v.at[pl.ds(s * GCH, GCH)],
                sem_g,
            )

        def o_desc(i):
            s = lax.rem(i, R)
            return pltpu.make_async_copy(
                rows_v.at[pl.ds(s * GCH, GCH)],
                out_hbm.at[pl.ds(base + i * GCH, GCH)],
                sem_o,
            )

        def step(j, _):
            @pl.when(j >= R)
            def _():
                o_desc(j - R).wait()

            @pl.when(j < nb)
            def _():
                g_desc(j).start()

            @pl.when(j >= Q)
            def _():
                i = j - Q
                g_desc(i).wait()
                s = lax.rem(i, R)
                for gg in range(GCH // LANES):
                    v = tok_v[pl.ds(i * GCH + gg * LANES, LANES)]
                    m = v < 0
                    rowids = s * GCH + gg * LANES + lane
                    for c in range(D):
                        plsc.store_scatter(
                            rows_v, [rowids, cols[c]], zeros, mask=m
                        )
                o_desc(i).start()

            return 0

        lax.fori_loop(0, nb + Q, step, 0)

        for r in range(R - Q):
            o_desc(nb - (R - Q) + r).wait()

    return k


def kernel(ecg_tokens, emb_table):
    B, L = ecg_tokens.shape
    V, D = emb_table.shape
    N = B * L
    k = _build(N, V, D)
    out = k(ecg_tokens.reshape(N), emb_table)
    return out.reshape(B, L, D)


# speedup vs baseline: 6.6781x; 3.6751x over previous
import functools

import jax
import jax.numpy as jnp
from jax import lax
from jax.experimental import pallas as pl
from jax.experimental.pallas import tpu as pltpu
from jax.experimental.pallas import tpu_sc as plsc

PAD = -2
SIGN = -(2**31)
MASK31 = 2**31 - 1


@functools.lru_cache(maxsize=None)
def _build(N, V, D):
    info = plsc.get_sparse_core_info()
    NC, NS, LANES = info.num_cores, info.num_subcores, info.num_lanes
    NW = NC * NS
    assert N % NW == 0
    per_w = N // NW
    GCH = 128
    assert per_w % GCH == 0
    nb = per_w // GCH
    S = 4
    RING = S * GCH

    mesh = plsc.VectorSubcoreMesh(core_axis_name="c", subcore_axis_name="s")

    @functools.partial(
        pl.kernel,
        out_type=jax.ShapeDtypeStruct((N, D), jnp.float32),
        mesh=mesh,
        compiler_params=pltpu.CompilerParams(
            needs_layout_passes=False, use_tc_tiling_on_sc=False
        ),
        scratch_types=[
            pltpu.VMEM((per_w,), jnp.int32),
            pltpu.VMEM((per_w,), jnp.int32),
            pltpu.VMEM((RING, D), jnp.float32),
            pltpu.VMEM((2 * GCH, D), jnp.float32),
            pltpu.SMEM((nb + 1,), jnp.int32),
            pltpu.SemaphoreType.DMA,
            pltpu.SemaphoreType.DMA,
        ],
    )
    def k(tok_hbm, table_hbm, out_hbm, meta_v, cidx_v, stage_v, oblk_v,
          pbnd_s, sem_g, sem_o):
        wid = lax.axis_index("s") * NC + lax.axis_index("c")
        base = wid * per_w
        lane = lax.iota(jnp.int32, LANES)
        zeros = jnp.zeros((LANES,), jnp.float32)
        izeros = jnp.zeros((LANES,), jnp.int32)
        cols = [jnp.full((LANES,), c, jnp.int32) for c in range(D)]

        pltpu.sync_copy(tok_hbm.at[pl.ds(base, per_w)], meta_v)

        def zero_cidx(i, _):
            cidx_v[pl.ds(i * LANES, LANES)] = izeros
            return 0

        lax.fori_loop(0, per_w // LANES, zero_cidx, 0)

        def prep(t, running):
            pbnd_s[t] = running
            for u in range(GCH // LANES):
                off = t * GCH + u * LANES
                v = meta_v[pl.ds(off, LANES)]
                m = v >= 0
                mi = m.astype(jnp.int32)
                cs = plsc.cumsum(mi)
                pexcl = running + cs - mi
                plsc.store_compressed(
                    cidx_v.at[pl.ds(running, LANES)], jnp.maximum(v, 0), mask=m
                )
                meta_v[pl.ds(off, LANES)] = pexcl | jnp.where(m, 0, SIGN)
                running = running + cs[15]
            return running

        nv = lax.fori_loop(0, nb, prep, jnp.int32(0))
        pbnd_s[nb] = nv
        nch = (nv + GCH - 1) // GCH

        def g_desc(c):
            s = lax.rem(c, S)
            return pltpu.make_async_copy(
                table_hbm.at[cidx_v.at[pl.ds(c * GCH, GCH)]],
                stage_v.at[pl.ds(s * GCH, GCH)],
                sem_g,
            )

        def o_desc(t):
            s = lax.rem(t, 2)
            return pltpu.make_async_copy(
                oblk_v.at[pl.ds(s * GCH, GCH)],
                out_hbm.at[pl.ds(base + t * GCH, GCH)],
                sem_o,
            )

        def block(t, carry):
            fired, waited = carry
            limit = jnp.minimum(nch, pbnd_s[t] // GCH + S)

            def fire_body(f):
                g_desc(f).start()
                return f + 1

            fired = lax.while_loop(lambda f: f < limit, fire_body, fired)

            need = (pbnd_s[t + 1] + GCH - 1) // GCH

            def wait_body(w):
                g_desc(w).wait()
                return w + 1

            waited = lax.while_loop(lambda w: w < need, wait_body, waited)

            @pl.when(t >= 2)
            def _():
                o_desc(t - 2).wait()

            obase = lax.rem(t, 2) * GCH

            def zfill(q, _):
                r = obase + q
                for c in range(D // LANES):
                    oblk_v[r, pl.ds(c * LANES, LANES)] = zeros
                return 0

            lax.fori_loop(0, GCH, zfill, 0)

            for gg in range(GCH // LANES):
                enc = meta_v[pl.ds(t * GCH + gg * LANES, LANES)]
                m = enc >= 0
                ring = lax.rem(enc & MASK31, RING)
                rowids = obase + gg * LANES + lane
                for c in range(D):
                    vals = plsc.load_gather(stage_v, [ring, cols[c]])
                    plsc.store_scatter(
                        oblk_v, [rowids, cols[c]], vals, mask=m
                    )

            o_desc(t).start()
            return fired, waited

        lax.fori_loop(0, nb, block, (jnp.int32(0), jnp.int32(0)))
        o_desc(nb - 2).wait()
        o_desc(nb - 1).wait()

    return k


def kernel(ecg_tokens, emb_table):
    B, L = ecg_tokens.shape
    V, D = emb_table.shape
    N = B * L
    k = _build(N, V, D)
    out = k(ecg_tokens.reshape(N), emb_table)
    return out.reshape(B, L, D)
